# baseline (speedup 1.0000x reference)
import functools

import jax
import jax.numpy as jnp
from jax import lax
from jax.experimental import pallas as pl
from jax.experimental.pallas import tpu as pltpu
from jax.experimental.pallas import tpu_sc as plsc

_NUM_CLASSES = 7
_DIM = 512
_NC = 2
_NS = 16
_NW = _NC * _NS
_LANES = 16


def _scale_body(t_ref, o_ref):
    o_ref[...] = t_ref[...] * (1.0 / _NUM_CLASSES)


@functools.cache
def _make_lookup(B):
    b_per_w = B // _NW
    mesh = plsc.VectorSubcoreMesh(core_axis_name="c", subcore_axis_name="s")

    @functools.partial(
        pl.kernel,
        out_type=jax.ShapeDtypeStruct((B, _DIM), jnp.float32),
        mesh=mesh,
        compiler_params=pltpu.CompilerParams(needs_layout_passes=False),
        scratch_types=[
            pltpu.VMEM((b_per_w,), jnp.int32),
            pltpu.SemaphoreType.DMA,
            pltpu.SemaphoreType.DMA,
        ],
    )
    def lookup_kernel(table_hbm, idx_hbm, out_hbm, idx_v, sem_in, sem_out):
        wid = lax.axis_index("s") * _NC + lax.axis_index("c")
        base = wid * b_per_w
        pltpu.async_copy(idx_hbm.at[pl.ds(base, b_per_w)], idx_v,
                         sem_in).wait()

        lanes = lax.broadcasted_iota(jnp.int32, (_LANES,), 0)

        def issue_row(j, carry):
            chunk = idx_v[pl.ds((j // _LANES) * _LANES, _LANES)]
            i = jnp.sum(jnp.where(lanes == lax.rem(j, _LANES), chunk, 0))
            pltpu.async_copy(table_hbm.at[i], out_hbm.at[base + j], sem_out)
            return carry

        lax.fori_loop(0, b_per_w, issue_row, 0)

        def drain_row(j, carry):
            pltpu.make_async_copy(table_hbm.at[0], out_hbm.at[base],
                                  sem_out).wait()
            return carry

        lax.fori_loop(0, b_per_w, drain_row, 0)

    return lookup_kernel


def kernel(x, de_class, clip_prompt):
    B = x.shape[0]
    scaled = pl.pallas_call(
        _scale_body,
        out_shape=jax.ShapeDtypeStruct((_NUM_CLASSES, _DIM), jnp.float32),
    )(clip_prompt)
    idx = de_class.astype(jnp.int32)
    return _make_lookup(B)(scaled, idx)


# speedup vs baseline: 3.6415x; 3.6415x over previous
import functools

import jax
import jax.numpy as jnp
from jax import lax
from jax.experimental import pallas as pl
from jax.experimental.pallas import tpu as pltpu
from jax.experimental.pallas import tpu_sc as plsc

_NUM_CLASSES = 7
_DIM = 512
_NC = 2
_NS = 16
_NW = _NC * _NS
_LANES = 16


@functools.cache
def _make_lookup(B):
    b_per_w = B // _NW
    mesh = plsc.VectorSubcoreMesh(core_axis_name="c", subcore_axis_name="s")

    @functools.partial(
        pl.kernel,
        out_type=jax.ShapeDtypeStruct((B, _DIM), jnp.float32),
        mesh=mesh,
        compiler_params=pltpu.CompilerParams(needs_layout_passes=False),
        scratch_types=[
            pltpu.VMEM((b_per_w,), jnp.int32),
            pltpu.VMEM((_NUM_CLASSES, _DIM), jnp.float32),
            pltpu.SemaphoreType.DMA,
            pltpu.SemaphoreType.DMA,
        ],
    )
    def lookup_kernel(table_hbm, idx_hbm, out_hbm, idx_v, tab_v, sem_in,
                      sem_out):
        wid = lax.axis_index("s") * _NC + lax.axis_index("c")
        base = wid * b_per_w
        cp_idx = pltpu.async_copy(idx_hbm.at[pl.ds(base, b_per_w)], idx_v,
                                  sem_in)
        tab_cps = []
        for k in range(_NUM_CLASSES):
            r = lax.rem(wid + k, _NUM_CLASSES)
            tab_cps.append(
                pltpu.async_copy(table_hbm.at[r], tab_v.at[r], sem_in))
        cp_idx.wait()
        for h in tab_cps:
            h.wait()

        scale = jnp.float32(1.0 / _NUM_CLASSES)

        def scale_row(r, carry):
            for c in range(_DIM // _LANES):
                sl = pl.ds(c * _LANES, _LANES)
                tab_v[r, sl] = tab_v[r, sl] * scale
            return carry

        lax.fori_loop(0, _NUM_CLASSES, scale_row, 0)

        lanes = lax.broadcasted_iota(jnp.int32, (_LANES,), 0)

        def issue_row(j, carry):
            chunk = idx_v[pl.ds((j // _LANES) * _LANES, _LANES)]
            i = jnp.sum(jnp.where(lanes == lax.rem(j, _LANES), chunk, 0))
            pltpu.async_copy(tab_v.at[i], out_hbm.at[base + j], sem_out)
            return carry

        lax.fori_loop(0, b_per_w, issue_row, 0, unroll=4)

        pltpu.make_async_copy(out_hbm.at[pl.ds(base, b_per_w)],
                              out_hbm.at[pl.ds(base, b_per_w)],
                              sem_out).wait()

    return lookup_kernel


def kernel(x, de_class, clip_prompt):
    B = x.shape[0]
    idx = de_class.astype(jnp.int32)
    return _make_lookup(B)(clip_prompt, idx)


# speedup vs baseline: 3.8240x; 1.0501x over previous
import functools

import jax
import jax.numpy as jnp
from jax import lax
from jax.experimental import pallas as pl
from jax.experimental.pallas import tpu as pltpu
from jax.experimental.pallas import tpu_sc as plsc

_NUM_CLASSES = 7
_DIM = 512
_NC = 2
_NS = 16
_NW = _NC * _NS
_LANES = 16


@functools.cache
def _make_lookup(B):
    b_per_w = B // _NW
    mesh = plsc.VectorSubcoreMesh(core_axis_name="c", subcore_axis_name="s")

    @functools.partial(
        pl.kernel,
        out_type=jax.ShapeDtypeStruct((B, _DIM), jnp.float32),
        mesh=mesh,
        compiler_params=pltpu.CompilerParams(needs_layout_passes=False),
        scratch_types=[
            pltpu.VMEM((b_per_w,), jnp.int32),
            pltpu.VMEM((_NUM_CLASSES, _DIM), jnp.float32),
            pltpu.VMEM_SHARED((_NUM_CLASSES, _DIM), jnp.float32),
            pltpu.SemaphoreType.DMA,
            pltpu.SemaphoreType.DMA,
        ],
    )
    def lookup_kernel(table_hbm, idx_hbm, out_hbm, idx_v, tab_v, tab_sh,
                      sem_in, sem_out):
        wid = lax.axis_index("s") * _NC + lax.axis_index("c")
        base = wid * b_per_w
        cp_idx = pltpu.async_copy(idx_hbm.at[pl.ds(base, b_per_w)], idx_v,
                                  sem_in)

        @pl.when(lax.axis_index("s") == 0)
        def _fetch_table():
            pltpu.sync_copy(table_hbm, tab_sh)

        plsc.subcore_barrier()
        pltpu.sync_copy(tab_sh, tab_v)
        cp_idx.wait()

        scale = jnp.float32(1.0 / _NUM_CLASSES)

        def scale_row(r, carry):
            for c in range(_DIM // _LANES):
                sl = pl.ds(c * _LANES, _LANES)
                tab_v[r, sl] = tab_v[r, sl] * scale
            return carry

        lax.fori_loop(0, _NUM_CLASSES, scale_row, 0)

        lanes = lax.broadcasted_iota(jnp.int32, (_LANES,), 0)

        def issue_row(j, carry):
            chunk = idx_v[pl.ds((j // _LANES) * _LANES, _LANES)]
            i = jnp.sum(jnp.where(lanes == lax.rem(j, _LANES), chunk, 0))
            pltpu.async_copy(tab_v.at[i], out_hbm.at[base + j], sem_out)
            return carry

        lax.fori_loop(0, b_per_w, issue_row, 0, unroll=4)

        pltpu.make_async_copy(out_hbm.at[pl.ds(base, b_per_w)],
                              out_hbm.at[pl.ds(base, b_per_w)],
                              sem_out).wait()

    return lookup_kernel


def kernel(x, de_class, clip_prompt):
    B = x.shape[0]
    idx = de_class.astype(jnp.int32)
    return _make_lookup(B)(clip_prompt, idx)
